# baseline (speedup 1.0000x reference)
import functools

import jax
import jax.numpy as jnp
from jax import lax
from jax.experimental import pallas as pl
from jax.experimental.pallas import tpu as pltpu
from jax.experimental.pallas import tpu_sc as plsc

_N = 400000
_N_FULL = 800000
_C = 128
_NSUB = 32
_W = 128
_NW = 100
_CHUNK = _NW * _W
_NPAD = _NSUB * _CHUNK
_PAD = _NPAD - _N
_NROWS = _NPAD // _W
_ZR = 448
_D = 4
_RREAL = _N // _W


_TAIL = _CHUNK - _PAD


def _src_base(r):
    dup = (_N - _TAIL) + ((r - _RREAL) % (_TAIL // _W)) * _W
    return jnp.where(r >= _RREAL, dup, r * _W)


def _sc_body(data_hbm, idx_hbm, out_hbm, idx_v, data_v, zero_v, lsem, ssem,
             zsem):
    wid = lax.axis_index("c") * 16 + lax.axis_index("s")
    row0 = wid * _NW

    zvec = jnp.zeros((16,), jnp.float32)

    @pl.loop(0, _ZR)
    def _(r):
        for c in range(_C // 16):
            zero_v[r, pl.ds(c * 16, 16)] = zvec

    pltpu.sync_copy(idx_hbm.at[row0], idx_v.at[0])
    s0 = idx_v[0, pl.ds(0, 16)][0]
    rn = jnp.minimum(row0 + _NW, _NROWS - 1)
    pltpu.sync_copy(idx_hbm.at[rn], idx_v.at[0])
    s1 = idx_v[0, pl.ds(0, 16)][0]
    zs = jnp.where(wid == 0, 0, s0)
    ze = jnp.where(wid == _NSUB - 1, _N_FULL, s1)

    nblk = (ze - zs) // _ZR

    @pl.loop(0, nblk)
    def _(t):
        pltpu.async_copy(zero_v, out_hbm.at[pl.ds(zs + t * _ZR, _ZR)], zsem)

    pltpu.async_copy(zero_v, out_hbm.at[pl.ds(ze - _ZR, _ZR)], zsem)

    for b in range(2):
        pltpu.async_copy(idx_hbm.at[row0 + b], idx_v.at[b], lsem.at[b])
        pltpu.async_copy(
            data_hbm.at[pl.ds(_src_base(row0 + b), _W)], data_v.at[b],
            lsem.at[b])

    @pl.loop(0, nblk + 1)
    def _(t):
        pltpu.make_async_copy(zero_v, out_hbm.at[pl.ds(zs, _ZR)],
                              zsem).wait()

    @pl.loop(0, _NW, step=_D)
    def _(w):
        for b in range(_D):
            ww = w + b
            b2 = (b + 2) % _D
            pltpu.make_async_copy(idx_hbm.at[row0], idx_v.at[b],
                                  lsem.at[b]).wait()
            pltpu.make_async_copy(data_hbm.at[pl.ds(0, _W)], data_v.at[b],
                                  lsem.at[b]).wait()
            pltpu.async_copy(data_v.at[b], out_hbm.at[idx_v.at[b]],
                             ssem.at[b])

            @pl.when(ww >= 2)
            def _():
                pltpu.make_async_copy(data_v.at[b2],
                                      out_hbm.at[idx_v.at[b2]],
                                      ssem.at[b2]).wait()

            @pl.when(ww + 2 < _NW)
            def _():
                r = row0 + ww + 2
                pltpu.async_copy(idx_hbm.at[r], idx_v.at[b2], lsem.at[b2])
                pltpu.async_copy(
                    data_hbm.at[pl.ds(_src_base(r), _W)], data_v.at[b2],
                    lsem.at[b2])

    for ww in (_NW - 2, _NW - 1):
        b = ww % _D
        pltpu.make_async_copy(data_v.at[b], out_hbm.at[idx_v.at[b]],
                              ssem.at[b]).wait()


@jax.jit
def _octree_pad(data_pad, idx2d):
    mesh = plsc.VectorSubcoreMesh(core_axis_name="c", subcore_axis_name="s")
    run = pl.kernel(
        _sc_body,
        out_type=jax.ShapeDtypeStruct((_N_FULL, _C), jnp.float32),
        mesh=mesh,
        scratch_types=[
            pltpu.VMEM((_D, _W), jnp.int32),
            pltpu.VMEM((_D, _W, _C), jnp.float32),
            pltpu.VMEM((_ZR, _C), jnp.float32),
            pltpu.SemaphoreType.DMA((_D,)),
            pltpu.SemaphoreType.DMA((_D,)),
            pltpu.SemaphoreType.DMA,
        ],
        compiler_params=pltpu.CompilerParams(use_tc_tiling_on_sc=False),
    )
    return run(data_pad, idx2d)


def kernel(data_in, octree):
    idx = octree.astype(jnp.int32)
    tail = idx[_N - _TAIL:]
    idx_pad = jnp.concatenate([idx] + [tail] * (_PAD // _TAIL))
    return _octree_pad(data_in, idx_pad.reshape(_NROWS, _W))


# speedup vs baseline: 1.0291x; 1.0291x over previous
import jax
import jax.numpy as jnp
from jax import lax
from jax.experimental import pallas as pl
from jax.experimental.pallas import tpu as pltpu
from jax.experimental.pallas import tpu_sc as plsc

_N = 400000
_N_FULL = 800000
_C = 128
_NSUB = 32
_W = 128
_NW = 100
_NROWS = _N // _W
_NW_LAST = _NROWS - (_NSUB - 1) * _NW
_ZR = 448
_D = 4


def _sc_body(data_hbm, idx_hbm, out_hbm, idx_v, data_v, zero_v, lsem, ssem,
             zsem):
    wid = lax.axis_index("c") * 16 + lax.axis_index("s")
    row0 = wid * _NW
    nw = jnp.where(wid == _NSUB - 1, _NW_LAST, _NW)

    zvec = jnp.zeros((16,), jnp.float32)

    @pl.loop(0, _ZR)
    def _(r):
        for c in range(_C // 16):
            zero_v[r, pl.ds(c * 16, 16)] = zvec

    pltpu.sync_copy(idx_hbm.at[row0], idx_v.at[0])
    s0 = idx_v[0, pl.ds(0, 16)][0]
    rn = jnp.minimum(row0 + _NW, _NROWS - 1)
    pltpu.sync_copy(idx_hbm.at[rn], idx_v.at[0])
    s1 = idx_v[0, pl.ds(0, 16)][0]
    zs = jnp.where(wid == 0, 0, s0)
    ze = jnp.where(wid == _NSUB - 1, _N_FULL, s1)

    nblk = (ze - zs) // _ZR

    @pl.loop(0, nblk)
    def _(t):
        pltpu.async_copy(zero_v, out_hbm.at[pl.ds(zs + t * _ZR, _ZR)], zsem)

    pltpu.async_copy(zero_v, out_hbm.at[pl.ds(ze - _ZR, _ZR)], zsem)

    for b in range(2):
        pltpu.async_copy(idx_hbm.at[row0 + b], idx_v.at[b], lsem.at[b])
        pltpu.async_copy(
            data_hbm.at[pl.ds((row0 + b) * _W, _W)], data_v.at[b],
            lsem.at[b])

    @pl.loop(0, nblk + 1)
    def _(t):
        pltpu.make_async_copy(zero_v, out_hbm.at[pl.ds(zs, _ZR)],
                              zsem).wait()

    @pl.loop(0, _NW, step=_D)
    def _(w):
        for b in range(_D):
            ww = w + b
            b2 = (b + 2) % _D

            @pl.when(ww < nw)
            def _():
                pltpu.make_async_copy(idx_hbm.at[row0], idx_v.at[b],
                                      lsem.at[b]).wait()
                pltpu.make_async_copy(data_hbm.at[pl.ds(0, _W)],
                                      data_v.at[b], lsem.at[b]).wait()
                pltpu.async_copy(data_v.at[b], out_hbm.at[idx_v.at[b]],
                                 ssem.at[b])

            @pl.when((ww >= 2) & (ww < nw))
            def _():
                pltpu.make_async_copy(data_v.at[b2],
                                      out_hbm.at[idx_v.at[b2]],
                                      ssem.at[b2]).wait()

            @pl.when(ww + 2 < nw)
            def _():
                r = row0 + ww + 2
                pltpu.async_copy(idx_hbm.at[r], idx_v.at[b2], lsem.at[b2])
                pltpu.async_copy(
                    data_hbm.at[pl.ds(r * _W, _W)], data_v.at[b2],
                    lsem.at[b2])

    for b in range(_D):
        @pl.when((lax.rem(nw - 2, _D) == b) | (lax.rem(nw - 1, _D) == b))
        def _():
            pltpu.make_async_copy(data_v.at[b], out_hbm.at[idx_v.at[b]],
                                  ssem.at[b]).wait()


@jax.jit
def _octree_pad(data_in, idx2d):
    mesh = plsc.VectorSubcoreMesh(core_axis_name="c", subcore_axis_name="s")
    run = pl.kernel(
        _sc_body,
        out_type=jax.ShapeDtypeStruct((_N_FULL, _C), jnp.float32),
        mesh=mesh,
        scratch_types=[
            pltpu.VMEM((_D, _W), jnp.int32),
            pltpu.VMEM((_D, _W, _C), jnp.float32),
            pltpu.VMEM((_ZR, _C), jnp.float32),
            pltpu.SemaphoreType.DMA((_D,)),
            pltpu.SemaphoreType.DMA((_D,)),
            pltpu.SemaphoreType.DMA,
        ],
        compiler_params=pltpu.CompilerParams(use_tc_tiling_on_sc=False),
    )
    return run(data_in, idx2d)


def kernel(data_in, octree):
    idx = octree.astype(jnp.int32)
    return _octree_pad(data_in, idx.reshape(_NROWS, _W))


# speedup vs baseline: 1.0421x; 1.0127x over previous
import jax
import jax.numpy as jnp
from jax import lax
from jax.experimental import pallas as pl
from jax.experimental.pallas import tpu as pltpu
from jax.experimental.pallas import tpu_sc as plsc

_N = 400000
_N_FULL = 800000
_C = 128
_NSUB = 32
_W = 128
_NW = 100
_NROWS = _N // _W
_NW_LAST = _NROWS - (_NSUB - 1) * _NW
_ZR = 448
_D = 4
_G = 8


def _sc_body(data_hbm, idx_hbm, out_hbm, idx_v, data_v, zero_v, gdone_s,
             lsem, ssem, zsem):
    wid = lax.axis_index("c") * 16 + lax.axis_index("s")
    row0 = wid * _NW
    nw = jnp.where(wid == _NSUB - 1, _NW_LAST, _NW)

    zvec = jnp.zeros((16,), jnp.float32)

    @pl.loop(0, _ZR)
    def _(r):
        for c in range(_C // 16):
            zero_v[r, pl.ds(c * 16, 16)] = zvec

    pltpu.sync_copy(idx_hbm.at[row0], idx_v.at[0])
    s0 = idx_v[0, pl.ds(0, 16)][0]
    rn = jnp.minimum(row0 + _NW, _NROWS - 1)
    pltpu.sync_copy(idx_hbm.at[rn], idx_v.at[0])
    s1 = idx_v[0, pl.ds(0, 16)][0]
    zs = jnp.where(wid == 0, 0, s0)
    ze = jnp.where(wid == _NSUB - 1, _N_FULL, s1)

    nblk = (ze - zs) // _ZR
    nb = nblk + 1
    glen = (nb + _G - 1) // _G

    def _cnt(g):
        return jnp.clip(nb - g * glen, 0, glen)

    for g in range(_G):
        @pl.loop(0, _cnt(g))
        def _(t):
            blk = g * glen + t
            start = jnp.where(blk == nblk, ze - _ZR, zs + blk * _ZR)
            pltpu.async_copy(zero_v, out_hbm.at[pl.ds(start, _ZR)],
                             zsem.at[g])

    for b in range(2):
        pltpu.async_copy(idx_hbm.at[row0 + b], idx_v.at[b], lsem.at[b])
        pltpu.async_copy(
            data_hbm.at[pl.ds((row0 + b) * _W, _W)], data_v.at[b],
            lsem.at[b])

    gdone_s[0] = 0

    def _drain_groups_below(req_g):
        gdone = gdone_s[0]
        for g in range(_G):
            @pl.when((g >= gdone) & (g < req_g))
            def _():
                @pl.loop(0, _cnt(g))
                def _(t):
                    pltpu.make_async_copy(zero_v,
                                          out_hbm.at[pl.ds(zs, _ZR)],
                                          zsem.at[g]).wait()
        gdone_s[0] = jnp.maximum(gdone, req_g)

    @pl.loop(0, _NW, step=_D)
    def _(w):
        for b in range(_D):
            ww = w + b
            b2 = (b + 2) % _D

            @pl.when(ww < nw)
            def _():
                pltpu.make_async_copy(idx_hbm.at[row0], idx_v.at[b],
                                      lsem.at[b]).wait()
                pltpu.make_async_copy(data_hbm.at[pl.ds(0, _W)],
                                      data_v.at[b], lsem.at[b]).wait()
                m = idx_v[b, pl.ds(_W - 16, 16)][15]
                req_b = jnp.minimum((m - zs) // _ZR + 1, nb)
                req_g = (req_b + glen - 1) // glen
                _drain_groups_below(req_g)
                pltpu.async_copy(data_v.at[b], out_hbm.at[idx_v.at[b]],
                                 ssem.at[b])

            @pl.when((ww >= 2) & (ww < nw))
            def _():
                pltpu.make_async_copy(data_v.at[b2],
                                      out_hbm.at[idx_v.at[b2]],
                                      ssem.at[b2]).wait()

            @pl.when(ww + 2 < nw)
            def _():
                r = row0 + ww + 2
                pltpu.async_copy(idx_hbm.at[r], idx_v.at[b2], lsem.at[b2])
                pltpu.async_copy(
                    data_hbm.at[pl.ds(r * _W, _W)], data_v.at[b2],
                    lsem.at[b2])

    _drain_groups_below(_G)

    for b in range(_D):
        @pl.when((lax.rem(nw - 2, _D) == b) | (lax.rem(nw - 1, _D) == b))
        def _():
            pltpu.make_async_copy(data_v.at[b], out_hbm.at[idx_v.at[b]],
                                  ssem.at[b]).wait()


@jax.jit
def _octree_pad(data_in, idx2d):
    mesh = plsc.VectorSubcoreMesh(core_axis_name="c", subcore_axis_name="s")
    run = pl.kernel(
        _sc_body,
        out_type=jax.ShapeDtypeStruct((_N_FULL, _C), jnp.float32),
        mesh=mesh,
        scratch_types=[
            pltpu.VMEM((_D, _W), jnp.int32),
            pltpu.VMEM((_D, _W, _C), jnp.float32),
            pltpu.VMEM((_ZR, _C), jnp.float32),
            pltpu.SMEM((1,), jnp.int32),
            pltpu.SemaphoreType.DMA((_D,)),
            pltpu.SemaphoreType.DMA((_D,)),
            pltpu.SemaphoreType.DMA((_G,)),
        ],
        compiler_params=pltpu.CompilerParams(use_tc_tiling_on_sc=False),
    )
    return run(data_in, idx2d)


def kernel(data_in, octree):
    idx = octree.astype(jnp.int32)
    return _octree_pad(data_in, idx.reshape(_NROWS, _W))
